# baseline (speedup 1.0000x reference)
import functools

import jax
import jax.numpy as jnp
from jax import lax
from jax.experimental import pallas as pl
from jax.experimental.pallas import tpu as pltpu
from jax.experimental.pallas import tpu_sc as plsc

B = 16384
H = 128
BINS = 32
NC, NS, L = 2, 16, 16
NW = NC * NS
BPW = B // NW
CHUNK = 128
NCHUNK = BPW // CHUNK
_ROWS = 2048


def _sc_body(p_hbm, t_hbm, pe_hbm, te_hbm, out_hbm,
             pv, tv, pi0, pi1, pi2, pi3, ti0, ti1, ti2, ti3,
             st0, st1, gsem, wsem):
    wid = lax.axis_index("s") * NC + lax.axis_index("c")
    base = wid * BPW
    off = wid * BINS
    pis = (pi0, pi1, pi2, pi3)
    tis = (ti0, ti1, ti2, ti3)
    sts = (st0, st1)

    pltpu.sync_copy(p_hbm.at[pl.ds(base, BPW)], pv)
    pltpu.sync_copy(t_hbm.at[pl.ds(base, BPW)], tv)

    for c in range(NCHUNK):
        for g in range(CHUNK // L):
            sl = pl.ds(c * CHUNK + g * L, L)
            dl = pl.ds(g * L, L)
            pb = jnp.clip(pv[sl], 0.0, 1.0) * float(BINS)
            pis[c][dl] = jnp.minimum(pb.astype(jnp.int32), BINS - 1) + off
            tb = jnp.clip(tv[sl], 0.0, 1.0) * float(BINS)
            tis[c][dl] = jnp.minimum(tb.astype(jnp.int32), BINS - 1) + off

    def fire_gather(c):
        st = sts[c % 2]
        return (pltpu.async_copy(pe_hbm.at[pis[c]],
                                 st.at[:, pl.ds(H, H)], gsem),
                pltpu.async_copy(te_hbm.at[tis[c]],
                                 st.at[:, pl.ds(2 * H, H)], gsem))

    def fire_write(c):
        r0 = base + c * CHUNK
        return (pltpu.async_copy(sts[c % 2], out_hbm.at[pl.ds(r0, CHUNK)],
                                 wsem),)

    gathers = {c: fire_gather(c) for c in range(min(2, NCHUNK))}
    writes = {}
    for c in range(NCHUNK):
        for cp in gathers.pop(c):
            cp.wait()
        writes[c] = fire_write(c)
        nxt = c + 2
        if nxt < NCHUNK:
            for cp in writes.pop(nxt - 2):
                cp.wait()
            gathers[nxt] = fire_gather(nxt)
    for ws in writes.values():
        for cp in ws:
            cp.wait()


_sc_gather = functools.partial(
    pl.kernel,
    out_type=jax.ShapeDtypeStruct((B, 3 * H), jnp.float32),
    mesh=plsc.VectorSubcoreMesh(core_axis_name="c", subcore_axis_name="s"),
    scratch_types=[
        pltpu.VMEM((BPW,), jnp.float32),
        pltpu.VMEM((BPW,), jnp.float32),
        pltpu.VMEM((CHUNK,), jnp.int32),
        pltpu.VMEM((CHUNK,), jnp.int32),
        pltpu.VMEM((CHUNK,), jnp.int32),
        pltpu.VMEM((CHUNK,), jnp.int32),
        pltpu.VMEM((CHUNK,), jnp.int32),
        pltpu.VMEM((CHUNK,), jnp.int32),
        pltpu.VMEM((CHUNK,), jnp.int32),
        pltpu.VMEM((CHUNK,), jnp.int32),
        pltpu.VMEM((CHUNK, 3 * H), jnp.float32),
        pltpu.VMEM((CHUNK, 3 * H), jnp.float32),
        pltpu.SemaphoreType.DMA,
        pltpu.SemaphoreType.DMA,
    ],
)(_sc_body)


def _proj_body(s_ref, wp_ref, p_ref, t_ref, buf_ref, o_ref):
    del buf_ref
    p = p_ref[...].reshape(1, _ROWS)
    t = t_ref[...].reshape(1, _ROWS)
    pc = jnp.clip(p, 0.0, 1.0)
    tc = jnp.clip(t, 0.0, 1.0)
    h0 = jnp.maximum(pc * s_ref[0] + tc * s_ref[2] + s_ref[4], 0.0)
    h1 = jnp.maximum(pc * s_ref[1] + tc * s_ref[3] + s_ref[5], 0.0)
    ones = jnp.ones((1, _ROWS), jnp.float32)
    hext = jnp.concatenate([h0, h1, ones], axis=0)
    w2e = wp_ref[0:3, :]
    proj = lax.dot_general(hext, w2e, (((0,), (0,)), ((), ())),
                           preferred_element_type=jnp.float32,
                           precision=lax.Precision.HIGHEST)
    o_ref[...] = proj


def kernel(pressure, temperature, w1, b1, w2, b2, p_emb, t_emb):
    s = jnp.concatenate([w1.reshape(-1), b1.reshape(-1)])
    wp = jnp.zeros((8, H), jnp.float32).at[0:2].set(w2).at[2].set(b2)
    pf = pressure.reshape(B // _ROWS, 1, _ROWS)
    tf = temperature.reshape(B // _ROWS, 1, _ROWS)

    pe_r = jnp.tile(p_emb, (NW, 1))
    te_r = jnp.tile(t_emb, (NW, 1))
    buf = _sc_gather(pressure, temperature, pe_r, te_r)

    out = pl.pallas_call(
        _proj_body,
        grid=(B // _ROWS,),
        in_specs=[
            pl.BlockSpec(memory_space=pltpu.SMEM),
            pl.BlockSpec((8, H), lambda i: (0, 0)),
            pl.BlockSpec((1, 1, _ROWS), lambda i: (i, 0, 0)),
            pl.BlockSpec((1, 1, _ROWS), lambda i: (i, 0, 0)),
            pl.BlockSpec(memory_space=pl.ANY),
        ],
        out_specs=pl.BlockSpec((_ROWS, H), lambda i: (i, 0)),
        out_shape=jax.ShapeDtypeStruct((B, 3 * H), jnp.float32),
        input_output_aliases={4: 0},
    )(s, wp, pf, tf, buf)
    return out


# speedup vs baseline: 1.0609x; 1.0609x over previous
import functools

import jax
import jax.numpy as jnp
from jax import lax
from jax.experimental import pallas as pl
from jax.experimental.pallas import tpu as pltpu
from jax.experimental.pallas import tpu_sc as plsc

B = 16384
H = 128
BINS = 32
NC, NS, L = 2, 16, 16
NW = NC * NS
BPW = B // NW
CHUNK = 128
NCHUNK = BPW // CHUNK
_ROWS = 2048


def _sc_body(p_hbm, t_hbm, pe_hbm, te_hbm, out_hbm,
             pv, tv, pi0, pi1, pi2, pi3, ti0, ti1, ti2, ti3,
             st0, st1, gsem, wsem):
    wid = lax.axis_index("s") * NC + lax.axis_index("c")
    base = wid * BPW
    off = wid * BINS
    pis = (pi0, pi1, pi2, pi3)
    tis = (ti0, ti1, ti2, ti3)
    sts = (st0, st1)

    pltpu.sync_copy(p_hbm.at[pl.ds(base, BPW)], pv)
    pltpu.sync_copy(t_hbm.at[pl.ds(base, BPW)], tv)

    for c in range(NCHUNK):
        for g in range(CHUNK // L):
            sl = pl.ds(c * CHUNK + g * L, L)
            dl = pl.ds(g * L, L)
            pb = jnp.clip(pv[sl], 0.0, 1.0) * float(BINS)
            pis[c][dl] = jnp.minimum(pb.astype(jnp.int32), BINS - 1) + off
            tb = jnp.clip(tv[sl], 0.0, 1.0) * float(BINS)
            tis[c][dl] = jnp.minimum(tb.astype(jnp.int32), BINS - 1) + off

    def fire_gather(c):
        st = sts[c % 2]
        return (pltpu.async_copy(pe_hbm.at[pis[c]],
                                 st.at[:, pl.ds(0, H)], gsem),
                pltpu.async_copy(te_hbm.at[tis[c]],
                                 st.at[:, pl.ds(H, H)], gsem))

    def fire_write(c):
        r0 = base + c * CHUNK
        return (pltpu.async_copy(
            sts[c % 2],
            out_hbm.at[pl.ds(r0, CHUNK), pl.ds(H, 2 * H)], wsem),)

    gathers = {c: fire_gather(c) for c in range(min(2, NCHUNK))}
    writes = {}
    for c in range(NCHUNK):
        for cp in gathers.pop(c):
            cp.wait()
        writes[c] = fire_write(c)
        nxt = c + 2
        if nxt < NCHUNK:
            for cp in writes.pop(nxt - 2):
                cp.wait()
            gathers[nxt] = fire_gather(nxt)
    for ws in writes.values():
        for cp in ws:
            cp.wait()


_sc_gather = functools.partial(
    pl.kernel,
    out_type=jax.ShapeDtypeStruct((B, 3 * H), jnp.float32),
    mesh=plsc.VectorSubcoreMesh(core_axis_name="c", subcore_axis_name="s"),
    scratch_types=[
        pltpu.VMEM((BPW,), jnp.float32),
        pltpu.VMEM((BPW,), jnp.float32),
        pltpu.VMEM((CHUNK,), jnp.int32),
        pltpu.VMEM((CHUNK,), jnp.int32),
        pltpu.VMEM((CHUNK,), jnp.int32),
        pltpu.VMEM((CHUNK,), jnp.int32),
        pltpu.VMEM((CHUNK,), jnp.int32),
        pltpu.VMEM((CHUNK,), jnp.int32),
        pltpu.VMEM((CHUNK,), jnp.int32),
        pltpu.VMEM((CHUNK,), jnp.int32),
        pltpu.VMEM((CHUNK, 2 * H), jnp.float32),
        pltpu.VMEM((CHUNK, 2 * H), jnp.float32),
        pltpu.SemaphoreType.DMA,
        pltpu.SemaphoreType.DMA,
    ],
)(_sc_body)


def _proj_body(s_ref, wp_ref, p_ref, t_ref, buf_ref, o_ref):
    del buf_ref
    p = p_ref[...].reshape(1, _ROWS)
    t = t_ref[...].reshape(1, _ROWS)
    pc = jnp.clip(p, 0.0, 1.0)
    tc = jnp.clip(t, 0.0, 1.0)
    h0 = jnp.maximum(pc * s_ref[0] + tc * s_ref[2] + s_ref[4], 0.0)
    h1 = jnp.maximum(pc * s_ref[1] + tc * s_ref[3] + s_ref[5], 0.0)
    ones = jnp.ones((1, _ROWS), jnp.float32)
    hext = jnp.concatenate([h0, h1, ones], axis=0)
    w2e = wp_ref[0:3, :]
    proj = lax.dot_general(hext, w2e, (((0,), (0,)), ((), ())),
                           preferred_element_type=jnp.float32,
                           precision=lax.Precision.HIGHEST)
    o_ref[...] = proj


def kernel(pressure, temperature, w1, b1, w2, b2, p_emb, t_emb):
    s = jnp.concatenate([w1.reshape(-1), b1.reshape(-1)])
    wp = jnp.zeros((8, H), jnp.float32).at[0:2].set(w2).at[2].set(b2)
    pf = pressure.reshape(B // _ROWS, 1, _ROWS)
    tf = temperature.reshape(B // _ROWS, 1, _ROWS)

    pe_r = jnp.tile(p_emb, (NW, 1))
    te_r = jnp.tile(t_emb, (NW, 1))
    buf = _sc_gather(pressure, temperature, pe_r, te_r)

    out = pl.pallas_call(
        _proj_body,
        grid=(B // _ROWS,),
        in_specs=[
            pl.BlockSpec(memory_space=pltpu.SMEM),
            pl.BlockSpec((8, H), lambda i: (0, 0)),
            pl.BlockSpec((1, 1, _ROWS), lambda i: (i, 0, 0)),
            pl.BlockSpec((1, 1, _ROWS), lambda i: (i, 0, 0)),
            pl.BlockSpec(memory_space=pl.ANY),
        ],
        out_specs=pl.BlockSpec((_ROWS, H), lambda i: (i, 0)),
        out_shape=jax.ShapeDtypeStruct((B, 3 * H), jnp.float32),
        input_output_aliases={4: 0},
    )(s, wp, pf, tf, buf)
    return out


# speedup vs baseline: 1.0757x; 1.0139x over previous
import functools

import jax
import jax.numpy as jnp
from jax import lax
from jax.experimental import pallas as pl
from jax.experimental.pallas import tpu as pltpu
from jax.experimental.pallas import tpu_sc as plsc

B = 16384
H = 128
BINS = 32
NC, NS, L = 2, 16, 16
NW = NC * NS
BPW = B // NW
CHUNK = 64
NCHUNK = BPW // CHUNK
NBUF = 4
GPC = CHUNK // L


def _sc_body(p_hbm, t_hbm, pe_hbm, te_hbm, prm_hbm, wex_hbm, out_hbm,
             pv, tv, prmv, wexv,
             pi0, pi1, pi2, pi3, pi4, pi5, pi6, pi7,
             ti0, ti1, ti2, ti3, ti4, ti5, ti6, ti7,
             st0, st1, st2, st3, gsem, wsem):
    wid = lax.axis_index("s") * NC + lax.axis_index("c")
    base = wid * BPW
    off = wid * BINS
    pis = (pi0, pi1, pi2, pi3, pi4, pi5, pi6, pi7)
    tis = (ti0, ti1, ti2, ti3, ti4, ti5, ti6, ti7)
    sts = (st0, st1, st2, st3)

    pltpu.sync_copy(p_hbm.at[pl.ds(base, BPW)], pv)
    pltpu.sync_copy(t_hbm.at[pl.ds(base, BPW)], tv)
    pltpu.sync_copy(prm_hbm, prmv)
    pltpu.sync_copy(wex_hbm, wexv)

    for c in range(NCHUNK):
        for g in range(GPC):
            sl = pl.ds(c * CHUNK + g * L, L)
            dl = pl.ds(g * L, L)
            pb = jnp.clip(pv[sl], 0.0, 1.0) * float(BINS)
            pis[c][dl] = jnp.minimum(pb.astype(jnp.int32), BINS - 1) + off
            tb = jnp.clip(tv[sl], 0.0, 1.0) * float(BINS)
            tis[c][dl] = jnp.minimum(tb.astype(jnp.int32), BINS - 1) + off

    s0, s1, s2, s3, s4, s5 = (prmv[pl.ds(i, 1)][0] for i in range(6))
    w2a = [wexv[pl.ds(k * L, L)] for k in range(H // L)]
    w2b = [wexv[pl.ds(H + k * L, L)] for k in range(H // L)]
    b2v = [wexv[pl.ds(2 * H + k * L, L)] for k in range(H // L)]
    zi = jnp.zeros((L,), jnp.int32)

    def fire_gather(c):
        st = sts[c % NBUF]
        return (pltpu.async_copy(pe_hbm.at[pis[c]],
                                 st.at[:, pl.ds(H, H)], gsem),
                pltpu.async_copy(te_hbm.at[tis[c]],
                                 st.at[:, pl.ds(2 * H, H)], gsem))

    def fire_write(c):
        r0 = base + c * CHUNK
        return (pltpu.async_copy(sts[c % NBUF],
                                 out_hbm.at[pl.ds(r0, CHUNK)], wsem),)

    def compute_head(c):
        st = sts[c % NBUF]

        def row(j, carry):
            pc = jnp.clip(pv[pl.ds(c * CHUNK + j, 1)][0], 0.0, 1.0)
            tc = jnp.clip(tv[pl.ds(c * CHUNK + j, 1)][0], 0.0, 1.0)
            a = jnp.maximum(pc * s0 + tc * s2 + s4, 0.0)
            b = jnp.maximum(pc * s1 + tc * s3 + s5, 0.0)
            for k in range(H // L):
                x = a * w2a[k] + b * w2b[k] + b2v[k]
                st[j, pl.ds(k * L, L)] = x
            return carry

        lax.fori_loop(0, CHUNK, row, 0)

    gathers = {c: fire_gather(c) for c in range(min(NBUF, NCHUNK))}
    writes = {}
    for c in range(NCHUNK):
        if c >= 1 and c + NBUF - 1 < NCHUNK:
            for cp in writes.pop(c - 1):
                cp.wait()
            gathers[c + NBUF - 1] = fire_gather(c + NBUF - 1)
        compute_head(c)
        for cp in gathers.pop(c):
            cp.wait()
        writes[c] = fire_write(c)
    for ws in writes.values():
        for cp in ws:
            cp.wait()


_sc_run = functools.partial(
    pl.kernel,
    out_type=jax.ShapeDtypeStruct((B, 3 * H), jnp.float32),
    mesh=plsc.VectorSubcoreMesh(core_axis_name="c", subcore_axis_name="s"),
    scratch_types=[
        pltpu.VMEM((BPW,), jnp.float32),
        pltpu.VMEM((BPW,), jnp.float32),
        pltpu.VMEM((16,), jnp.float32),
        pltpu.VMEM((3 * H,), jnp.float32),
    ] + [pltpu.VMEM((CHUNK,), jnp.int32)] * (2 * NCHUNK) + [
        pltpu.VMEM((CHUNK, 3 * H), jnp.float32),
        pltpu.VMEM((CHUNK, 3 * H), jnp.float32),
        pltpu.VMEM((CHUNK, 3 * H), jnp.float32),
        pltpu.VMEM((CHUNK, 3 * H), jnp.float32),
        pltpu.SemaphoreType.DMA,
        pltpu.SemaphoreType.DMA,
    ],
)(_sc_body)


def kernel(pressure, temperature, w1, b1, w2, b2, p_emb, t_emb):
    prm = jnp.zeros((16,), jnp.float32)
    prm = prm.at[0:4].set(w1.reshape(-1)).at[4:6].set(b1.reshape(-1))
    wex = jnp.concatenate([w2[0], w2[1], b2]).astype(jnp.float32)

    pe_r = jnp.tile(p_emb, (NW, 1))
    te_r = jnp.tile(t_emb, (NW, 1))
    return _sc_run(pressure, temperature, pe_r, te_r, prm, wex)
